# baseline (speedup 1.0000x reference)
import functools

import jax
import jax.numpy as jnp
from jax import lax
from jax.experimental import pallas as pl
from jax.experimental.pallas import tpu as pltpu
from jax.experimental.pallas import tpu_sc as plsc

T = 32768
D = 768
E = 64
K = 8
BT = 4096
NW = 32
TW = T // NW
GT = 512


def _logits_kernel(x_ref, wt_ref, lt_ref):
    x = x_ref[...]
    wt = wt_ref[...]
    lt_ref[...] = jnp.dot(x, wt, preferred_element_type=jnp.float32)


def _merge(ak, av, bk, bv, ii, shift_idx):
    bsk = bk.at[shift_idx].get(mode="promise_in_bounds")
    bsv = bv.at[shift_idx].get(mode="promise_in_bounds")
    ck = jnp.where(ii < 8, ak, bsk)
    cv = jnp.where(ii < 8, av, bsv)
    return plsc.sort_key_val(ck, cv, descending=True)


def _sc_topk(lt_hbm, idx_hbm, w_hbm, buf, idxb, wb):
    wid = lax.axis_index("s") * 2 + lax.axis_index("c")
    tok0 = wid * TW
    ii = lax.iota(jnp.int32, 16)
    shift_idx = jnp.maximum(ii - 8, 0)

    def group(g, _):
        gbase = tok0 + g * GT
        pltpu.sync_copy(lt_hbm.at[pl.ds(gbase * E, GT * E)], buf)

        def token(t, _):
            ks = []
            vs = []
            for q in range(4):
                rows = ii + 16 * q
                kq = buf[pl.ds(t * E + 16 * q, 16)]
                sk, sv = plsc.sort_key_val(kq, rows, descending=True)
                ks.append(sk)
                vs.append(sv)
            mk0, mv0 = _merge(ks[0], vs[0], ks[1], vs[1], ii, shift_idx)
            mk1, mv1 = _merge(ks[2], vs[2], ks[3], vs[3], ii, shift_idx)
            mk, mv = _merge(mk0, mv0, mk1, mv1, ii, shift_idx)

            m0 = jnp.max(mk)
            e = jnp.exp(mk - m0)
            es = jnp.where(ii < 8, e, 0.0)
            s = jnp.sum(es)
            w = e / jnp.broadcast_to(s, (16,))

            idxb[pl.ds(t * 16, 16)] = mv
            wb[pl.ds(t * 16, 16)] = w
            return ()

        lax.fori_loop(0, GT, token, (), unroll=4)
        pltpu.sync_copy(idxb, idx_hbm.at[pl.ds(gbase * 16, GT * 16)])
        pltpu.sync_copy(wb, w_hbm.at[pl.ds(gbase * 16, GT * 16)])
        return ()

    lax.fori_loop(0, TW // GT, group, ())


@jax.jit
def kernel(hidden_states, wg_weight):
    wt = wg_weight.astype(jnp.float32).T
    x = hidden_states.astype(jnp.float32)

    lt = pl.pallas_call(
        _logits_kernel,
        grid=(T // BT,),
        in_specs=[
            pl.BlockSpec((BT, D), lambda i: (i, 0)),
            pl.BlockSpec((D, E), lambda i: (0, 0)),
        ],
        out_specs=pl.BlockSpec((BT, E), lambda i: (i, 0)),
        out_shape=jax.ShapeDtypeStruct((T, E), jnp.float32),
    )(x, wt)

    mesh = plsc.VectorSubcoreMesh(core_axis_name="c", subcore_axis_name="s")
    idx16, w16 = functools.partial(
        pl.kernel,
        mesh=mesh,
        compiler_params=pltpu.CompilerParams(needs_layout_passes=False),
        out_type=[
            jax.ShapeDtypeStruct((T * 16,), jnp.int32),
            jax.ShapeDtypeStruct((T * 16,), jnp.float32),
        ],
        scratch_types=[
            pltpu.VMEM((GT * E,), jnp.float32),
            pltpu.VMEM((GT * 16,), jnp.int32),
            pltpu.VMEM((GT * 16,), jnp.float32),
        ],
    )(_sc_topk)(lt.reshape(T * E))

    idx = idx16.reshape(T, 16)[:, :K]
    w = w16.reshape(T, 16)[:, :K]
    return idx, w.astype(hidden_states.dtype)


# speedup vs baseline: 2.6651x; 2.6651x over previous
import jax
import jax.numpy as jnp
from jax.experimental import pallas as pl

T = 32768
D = 768
E = 64
K = 8
BT = 4096
CHT = 512

NEG_INF = float("-inf")


def _gate_kernel(x_ref, wg_ref, idx_ref, w_ref):
    wg = wg_ref[...]
    subl = jax.lax.broadcasted_iota(jnp.int32, (E, CHT), 0).astype(jnp.float32)
    for c in range(BT // CHT):
        rows = pl.ds(c * CHT, CHT)
        x = x_ref[rows, :]
        lt = jax.lax.dot_general(
            wg, x, (((1,), (1,)), ((), ())),
            preferred_element_type=jnp.float32)

        vals = []
        work = lt
        for k in range(K):
            m = jnp.max(work, axis=0, keepdims=True)
            vals.append(m)
            if k + 1 < K:
                work = jnp.where(work == m, NEG_INF, work)

        idxs = []
        for k in range(K):
            hit = lt == vals[k]
            idxs.append(jnp.min(jnp.where(hit, subl, jnp.float32(E)),
                                axis=0, keepdims=True))

        vt = jnp.concatenate(vals, axis=0)
        it = jnp.concatenate(idxs, axis=0)
        e = jnp.exp(vt - vt[0:1, :])
        w = e * (1.0 / jnp.sum(e, axis=0, keepdims=True))
        idx_ref[rows, :] = it.T.astype(jnp.int32)
        w_ref[rows, :] = w.T


@jax.jit
def kernel(hidden_states, wg_weight):
    wg = wg_weight.astype(jnp.float32)
    x = hidden_states.astype(jnp.float32)
    grid = (T // BT,)
    idx, w = pl.pallas_call(
        _gate_kernel,
        grid=grid,
        in_specs=[
            pl.BlockSpec((BT, D), lambda i: (i, 0)),
            pl.BlockSpec((E, D), lambda i: (0, 0)),
        ],
        out_specs=[
            pl.BlockSpec((BT, K), lambda i: (i, 0)),
            pl.BlockSpec((BT, K), lambda i: (i, 0)),
        ],
        out_shape=[
            jax.ShapeDtypeStruct((T, K), jnp.int32),
            jax.ShapeDtypeStruct((T, K), jnp.float32),
        ],
    )(x, wg)
    return idx, w.astype(hidden_states.dtype)
